# baseline (speedup 1.0000x reference)
import functools

import numpy as np
import jax
import jax.numpy as jnp
from jax import lax
from jax.experimental import pallas as pl
from jax.experimental.pallas import tpu as pltpu
from jax.experimental.pallas import tpu_sc as plsc

N = 10000
D = 256
E = 160000
H = 128
NT = 16
NCHUNK = 100
CEDGE = 100
HCH = NCHUNK // 2
EPAD = NT * NCHUNK * CEDGE
NP_ = 10240
ROWS_PER_TILE = NP_ // NT
RB = 1000
BN_INV = float(1.0 / np.sqrt(1.0 + 1e-5))

_sc_mesh = plsc.VectorSubcoreMesh(core_axis_name="c", subcore_axis_name="s")


@functools.partial(
    pl.kernel,
    out_type=jax.ShapeDtypeStruct((2 * NP_, H), jnp.float32),
    mesh=_sc_mesh,
    scratch_types=[
        pltpu.VMEM_SHARED((NP_, H), jnp.float32),
        pltpu.VMEM((HCH, CEDGE), jnp.int32),
        pltpu.VMEM((HCH, CEDGE), jnp.int32),
        pltpu.VMEM((CEDGE, H), jnp.float32),
        pltpu.VMEM((CEDGE, H), jnp.float32),
        pltpu.SemaphoreType.DMA,
        pltpu.SemaphoreType.DMA,
    ],
)
def _sc_aggregate(x_hbm, src_hbm, dst_hbm, zeros_hbm, out_hbm,
                  acc, src_v, dst_v, rows0, rows1, sem0, sem1):
    c = lax.axis_index("c")
    s = lax.axis_index("s")
    row0 = s * ROWS_PER_TILE
    w = c * NT + s
    pltpu.sync_copy(zeros_hbm, acc.at[pl.ds(row0, ROWS_PER_TILE)])
    plsc.subcore_barrier()

    for h in range(2):
        pltpu.sync_copy(src_hbm.at[2 * w + h], src_v)
        pltpu.sync_copy(dst_hbm.at[2 * s + h], dst_v)
        pltpu.async_copy(x_hbm.at[src_v.at[0]], rows0, sem0)

        def pair(p, carry):
            j0 = 2 * p
            pltpu.async_copy(x_hbm.at[src_v.at[j0 + 1]], rows1, sem1)
            pltpu.make_async_copy(x_hbm.at[src_v.at[j0]], rows0, sem0).wait()
            pltpu.sync_copy(rows0, acc.at[dst_v.at[j0]], add=True)

            @pl.when(p < HCH // 2 - 1)
            def _():
                pltpu.async_copy(x_hbm.at[src_v.at[j0 + 2]], rows0, sem0)

            pltpu.make_async_copy(x_hbm.at[src_v.at[j0 + 1]], rows1,
                                  sem1).wait()
            pltpu.sync_copy(rows1, acc.at[dst_v.at[j0 + 1]], add=True)
            return carry

        lax.fori_loop(0, HCH // 2, pair, 0)
    plsc.subcore_barrier()
    pltpu.sync_copy(acc.at[pl.ds(row0, ROWS_PER_TILE)],
                    out_hbm.at[pl.ds(c * NP_ + row0, ROWS_PER_TILE)])


def _mlp_body(agg_ref, x_ref, w1_ref, b1_ref, w2_ref, b2_ref, g_ref, be_ref,
              out_ref, *, final, xfirst):
    if xfirst:
        xl = x_ref[:, 0, :]
        xr = x_ref[:, 1, :]
    else:
        xl = x_ref[0]
        xr = x_ref[1]
    hl = agg_ref[0] + xl
    hr = agg_ref[1] + xr
    bf = jnp.bfloat16
    u = (jnp.dot(hl.astype(bf), w1_ref[:H, :].astype(bf),
                 preferred_element_type=jnp.float32)
         + jnp.dot(hr.astype(bf), w1_ref[H:, :].astype(bf),
                   preferred_element_type=jnp.float32)
         + b1_ref[0])
    u = jnp.maximum(u, 0.0)
    y = jnp.dot(u.astype(bf), w2_ref[...].astype(bf),
                preferred_element_type=jnp.float32) + b2_ref[0]
    y = jnp.maximum(y, 0.0)
    y = y * (g_ref[0] * BN_INV) + be_ref[0]
    if final:
        out_ref[...] = y
    else:
        out_ref[0] = y[:, :H]
        out_ref[1] = y[:, H:]


def _make_mlp(final, xfirst=False):
    stacked = pl.BlockSpec((2, RB, H), lambda i: (0, i, 0))
    full = pl.BlockSpec((D, D), lambda i: (0, 0))
    vec = pl.BlockSpec((1, D), lambda i: (0, 0))
    xspec = pl.BlockSpec((RB, 2, H), lambda i: (i, 0, 0)) if xfirst else stacked
    if final:
        out_spec = pl.BlockSpec((RB, D), lambda i: (i, 0))
        out_shape = jax.ShapeDtypeStruct((N, D), jnp.float32)
    else:
        out_spec = stacked
        out_shape = jax.ShapeDtypeStruct((2, N, H), jnp.float32)
    return pl.pallas_call(
        functools.partial(_mlp_body, final=final, xfirst=xfirst),
        grid=(N // RB,),
        in_specs=[stacked, xspec, full, vec, full, vec, vec, vec],
        out_specs=out_spec,
        out_shape=out_shape,
    )


_mlp_first = _make_mlp(final=False, xfirst=True)
_mlp_stacked = _make_mlp(final=False)
_mlp_final = _make_mlp(final=True)


def kernel(x, adj_t,
           W1_0, b1_0, W2_0, b2_0, g_0, be_0,
           W1_1, b1_1, W2_1, b2_1, g_1, be_1,
           W1_2, b1_2, W2_2, b2_2, g_2, be_2):
    params = [
        (W1_0, b1_0, W2_0, b2_0, g_0, be_0),
        (W1_1, b1_1, W2_1, b2_1, g_1, be_1),
        (W1_2, b1_2, W2_2, b2_2, g_2, be_2),
    ]
    npad_t = (EPAD - E) // NT
    src2d = adj_t[0].reshape(NT, E // NT)
    dst2d = adj_t[1].reshape(NT, E // NT)
    if npad_t:
        dsrc = jnp.zeros((NT, npad_t), jnp.int32)
        ddst = jnp.broadcast_to(
            N + jnp.arange(npad_t, dtype=jnp.int32), (NT, npad_t))
        src2d = jnp.concatenate([src2d, dsrc], axis=1)
        dst2d = jnp.concatenate([dst2d, ddst], axis=1)
    src16 = src2d.reshape(NT, NCHUNK, CEDGE)
    src_all = jnp.concatenate([src16, src16 + N], axis=0)
    src_all = src_all.reshape(2 * NT * 2, HCH, CEDGE)
    src0_all = jnp.concatenate([2 * src16, 2 * src16 + 1], axis=0)
    src0_all = src0_all.reshape(2 * NT * 2, HCH, CEDGE)
    dst_all = dst2d.reshape(NT * 2, HCH, CEDGE)
    zeros = jnp.zeros((ROWS_PER_TILE, H), jnp.float32)
    x2 = None
    for i in range(3):
        xs = x.reshape(2 * N, H) if i == 0 else x2
        srcs = src0_all if i == 0 else src_all
        agg3 = _sc_aggregate(xs, srcs, dst_all, zeros).reshape(2, NP_, H)
        x3 = x.reshape(N, 2, H) if i == 0 else x2.reshape(2, N, H)
        p = [params[i][0], params[i][1].reshape(1, D),
             params[i][2], params[i][3].reshape(1, D),
             params[i][4].reshape(1, D), params[i][5].reshape(1, D)]
        if i == 0:
            x2 = _mlp_first(agg3, x3, *p).reshape(2 * N, H)
        elif i == 1:
            x2 = _mlp_stacked(agg3, x3, *p).reshape(2 * N, H)
        else:
            return _mlp_final(agg3, x3, *p)


# speedup vs baseline: 1.0009x; 1.0009x over previous
import functools

import numpy as np
import jax
import jax.numpy as jnp
from jax import lax
from jax.experimental import pallas as pl
from jax.experimental.pallas import tpu as pltpu
from jax.experimental.pallas import tpu_sc as plsc

N = 10000
D = 256
E = 160000
H = 128
NT = 16
NCHUNK = 100
CEDGE = 100
HCH = NCHUNK // 2
EPAD = NT * NCHUNK * CEDGE
NP_ = 10240
ROWS_PER_TILE = NP_ // NT
RB = 1000
BN_INV = float(1.0 / np.sqrt(1.0 + 1e-5))

_sc_mesh = plsc.VectorSubcoreMesh(core_axis_name="c", subcore_axis_name="s")


@functools.partial(
    pl.kernel,
    out_type=jax.ShapeDtypeStruct((2 * NP_, H), jnp.float32),
    mesh=_sc_mesh,
    scratch_types=[
        pltpu.VMEM_SHARED((NP_, H), jnp.float32),
        pltpu.VMEM((HCH, CEDGE), jnp.int32),
        pltpu.VMEM((HCH, CEDGE), jnp.int32),
        pltpu.VMEM((CEDGE, H), jnp.float32),
        pltpu.VMEM((CEDGE, H), jnp.float32),
        pltpu.SemaphoreType.DMA,
        pltpu.SemaphoreType.DMA,
    ],
)
def _sc_aggregate(x_hbm, src_hbm, dst_hbm, zeros_hbm, out_hbm,
                  acc, src_v, dst_v, rows0, rows1, sem0, sem1):
    c = lax.axis_index("c")
    s = lax.axis_index("s")
    row0 = s * ROWS_PER_TILE
    w = c * NT + s
    pltpu.sync_copy(zeros_hbm, acc.at[pl.ds(row0, ROWS_PER_TILE)])
    plsc.subcore_barrier()

    for h in range(2):
        pltpu.sync_copy(src_hbm.at[2 * w + h], src_v)
        pltpu.sync_copy(dst_hbm.at[2 * s + h], dst_v)
        pltpu.async_copy(x_hbm.at[src_v.at[0]], rows0, sem0)

        def pair(p, carry):
            j0 = 2 * p
            pltpu.async_copy(x_hbm.at[src_v.at[j0 + 1]], rows1, sem1)
            pltpu.make_async_copy(x_hbm.at[src_v.at[j0]], rows0, sem0).wait()
            pltpu.sync_copy(rows0, acc.at[dst_v.at[j0]], add=True)

            @pl.when(p < HCH // 2 - 1)
            def _():
                pltpu.async_copy(x_hbm.at[src_v.at[j0 + 2]], rows0, sem0)

            pltpu.make_async_copy(x_hbm.at[src_v.at[j0 + 1]], rows1,
                                  sem1).wait()
            pltpu.sync_copy(rows1, acc.at[dst_v.at[j0 + 1]], add=True)
            return carry

        lax.fori_loop(0, HCH // 2, pair, 0)
    plsc.subcore_barrier()
    pltpu.sync_copy(acc.at[pl.ds(row0, ROWS_PER_TILE)],
                    out_hbm.at[pl.ds(c * NP_ + row0, ROWS_PER_TILE)])


def _mlp_body(agg_ref, x_ref, w1_ref, b1_ref, w2_ref, b2_ref, g_ref, be_ref,
              out_ref, *, final, xfirst):
    if xfirst:
        xl = x_ref[:, 0, :]
        xr = x_ref[:, 1, :]
    else:
        xl = x_ref[0]
        xr = x_ref[1]
    hl = agg_ref[0] + xl
    hr = agg_ref[1] + xr
    u = (jnp.dot(hl, w1_ref[:H, :], preferred_element_type=jnp.float32)
         + jnp.dot(hr, w1_ref[H:, :], preferred_element_type=jnp.float32)
         + b1_ref[0])
    u = jnp.maximum(u, 0.0)
    y = jnp.dot(u, w2_ref[...], preferred_element_type=jnp.float32) + b2_ref[0]
    y = jnp.maximum(y, 0.0)
    y = y * (g_ref[0] * BN_INV) + be_ref[0]
    if final:
        out_ref[...] = y
    else:
        out_ref[0] = y[:, :H]
        out_ref[1] = y[:, H:]


def _make_mlp(final, xfirst=False):
    stacked = pl.BlockSpec((2, RB, H), lambda i: (0, i, 0))
    full = pl.BlockSpec((D, D), lambda i: (0, 0))
    vec = pl.BlockSpec((1, D), lambda i: (0, 0))
    xspec = pl.BlockSpec((RB, 2, H), lambda i: (i, 0, 0)) if xfirst else stacked
    if final:
        out_spec = pl.BlockSpec((RB, D), lambda i: (i, 0))
        out_shape = jax.ShapeDtypeStruct((N, D), jnp.float32)
    else:
        out_spec = stacked
        out_shape = jax.ShapeDtypeStruct((2, N, H), jnp.float32)
    return pl.pallas_call(
        functools.partial(_mlp_body, final=final, xfirst=xfirst),
        grid=(N // RB,),
        in_specs=[stacked, xspec, full, vec, full, vec, vec, vec],
        out_specs=out_spec,
        out_shape=out_shape,
    )


_mlp_first = _make_mlp(final=False, xfirst=True)
_mlp_stacked = _make_mlp(final=False)
_mlp_final = _make_mlp(final=True)


def kernel(x, adj_t,
           W1_0, b1_0, W2_0, b2_0, g_0, be_0,
           W1_1, b1_1, W2_1, b2_1, g_1, be_1,
           W1_2, b1_2, W2_2, b2_2, g_2, be_2):
    params = [
        (W1_0, b1_0, W2_0, b2_0, g_0, be_0),
        (W1_1, b1_1, W2_1, b2_1, g_1, be_1),
        (W1_2, b1_2, W2_2, b2_2, g_2, be_2),
    ]
    npad_t = (EPAD - E) // NT
    src2d = adj_t[0].reshape(NT, E // NT)
    dst2d = adj_t[1].reshape(NT, E // NT)
    if npad_t:
        dsrc = jnp.zeros((NT, npad_t), jnp.int32)
        ddst = jnp.broadcast_to(
            N + jnp.arange(npad_t, dtype=jnp.int32), (NT, npad_t))
        src2d = jnp.concatenate([src2d, dsrc], axis=1)
        dst2d = jnp.concatenate([dst2d, ddst], axis=1)
    src16 = src2d.reshape(NT, NCHUNK, CEDGE)
    src_all = jnp.concatenate([src16, src16 + N], axis=0)
    src_all = src_all.reshape(2 * NT * 2, HCH, CEDGE)
    src0_all = jnp.concatenate([2 * src16, 2 * src16 + 1], axis=0)
    src0_all = src0_all.reshape(2 * NT * 2, HCH, CEDGE)
    dst_all = dst2d.reshape(NT * 2, HCH, CEDGE)
    zeros = jnp.zeros((ROWS_PER_TILE, H), jnp.float32)
    x2 = None
    for i in range(3):
        xs = x.reshape(2 * N, H) if i == 0 else x2
        srcs = src0_all if i == 0 else src_all
        agg3 = _sc_aggregate(xs, srcs, dst_all, zeros).reshape(2, NP_, H)
        x3 = x.reshape(N, 2, H) if i == 0 else x2.reshape(2, N, H)
        p = [params[i][0], params[i][1].reshape(1, D),
             params[i][2], params[i][3].reshape(1, D),
             params[i][4].reshape(1, D), params[i][5].reshape(1, D)]
        if i == 0:
            x2 = _mlp_first(agg3, x3, *p).reshape(2 * N, H)
        elif i == 1:
            x2 = _mlp_stacked(agg3, x3, *p).reshape(2 * N, H)
        else:
            return _mlp_final(agg3, x3, *p)


# speedup vs baseline: 1.0150x; 1.0141x over previous
import functools

import numpy as np
import jax
import jax.numpy as jnp
from jax import lax
from jax.experimental import pallas as pl
from jax.experimental.pallas import tpu as pltpu
from jax.experimental.pallas import tpu_sc as plsc

N = 10000
D = 256
E = 160000
H = 128
NT = 16
NCHUNK = 100
CEDGE = 100
HCH = NCHUNK // 2
EPAD = NT * NCHUNK * CEDGE
NP_ = 10240
ROWS_PER_TILE = NP_ // NT
RB = 2000
BN_INV = float(1.0 / np.sqrt(1.0 + 1e-5))

_sc_mesh = plsc.VectorSubcoreMesh(core_axis_name="c", subcore_axis_name="s")


@functools.partial(
    pl.kernel,
    out_type=jax.ShapeDtypeStruct((2 * NP_, H), jnp.float32),
    mesh=_sc_mesh,
    scratch_types=[
        pltpu.VMEM_SHARED((NP_, H), jnp.float32),
        pltpu.VMEM((HCH, CEDGE), jnp.int32),
        pltpu.VMEM((HCH, CEDGE), jnp.int32),
        pltpu.VMEM((CEDGE, H), jnp.float32),
        pltpu.VMEM((CEDGE, H), jnp.float32),
        pltpu.SemaphoreType.DMA,
        pltpu.SemaphoreType.DMA,
    ],
)
def _sc_aggregate(x_hbm, src_hbm, dst_hbm, zeros_hbm, out_hbm,
                  acc, src_v, dst_v, rows0, rows1, sem0, sem1):
    c = lax.axis_index("c")
    s = lax.axis_index("s")
    row0 = s * ROWS_PER_TILE
    w = c * NT + s
    pltpu.sync_copy(zeros_hbm, acc.at[pl.ds(row0, ROWS_PER_TILE)])
    plsc.subcore_barrier()

    for h in range(2):
        pltpu.sync_copy(src_hbm.at[2 * w + h], src_v)
        pltpu.sync_copy(dst_hbm.at[2 * s + h], dst_v)
        pltpu.async_copy(x_hbm.at[src_v.at[0]], rows0, sem0)

        def pair(p, carry):
            j0 = 2 * p
            pltpu.async_copy(x_hbm.at[src_v.at[j0 + 1]], rows1, sem1)
            pltpu.make_async_copy(x_hbm.at[src_v.at[j0]], rows0, sem0).wait()
            pltpu.sync_copy(rows0, acc.at[dst_v.at[j0]], add=True)

            @pl.when(p < HCH // 2 - 1)
            def _():
                pltpu.async_copy(x_hbm.at[src_v.at[j0 + 2]], rows0, sem0)

            pltpu.make_async_copy(x_hbm.at[src_v.at[j0 + 1]], rows1,
                                  sem1).wait()
            pltpu.sync_copy(rows1, acc.at[dst_v.at[j0 + 1]], add=True)
            return carry

        lax.fori_loop(0, HCH // 2, pair, 0)
    plsc.subcore_barrier()
    pltpu.sync_copy(acc.at[pl.ds(row0, ROWS_PER_TILE)],
                    out_hbm.at[pl.ds(c * NP_ + row0, ROWS_PER_TILE)])


def _mlp_body(agg_ref, x_ref, w1_ref, b1_ref, w2_ref, b2_ref, g_ref, be_ref,
              out_ref, *, final, xfirst):
    if xfirst:
        xl = x_ref[:, 0, :]
        xr = x_ref[:, 1, :]
    else:
        xl = x_ref[0]
        xr = x_ref[1]
    hl = agg_ref[0] + xl
    hr = agg_ref[1] + xr
    u = (jnp.dot(hl, w1_ref[:H, :], preferred_element_type=jnp.float32)
         + jnp.dot(hr, w1_ref[H:, :], preferred_element_type=jnp.float32)
         + b1_ref[0])
    u = jnp.maximum(u, 0.0)
    y = jnp.dot(u, w2_ref[...], preferred_element_type=jnp.float32) + b2_ref[0]
    y = jnp.maximum(y, 0.0)
    y = y * (g_ref[0] * BN_INV) + be_ref[0]
    if final:
        out_ref[...] = y
    else:
        out_ref[0] = y[:, :H]
        out_ref[1] = y[:, H:]


def _make_mlp(final, xfirst=False):
    stacked = pl.BlockSpec((2, RB, H), lambda i: (0, i, 0))
    full = pl.BlockSpec((D, D), lambda i: (0, 0))
    vec = pl.BlockSpec((1, D), lambda i: (0, 0))
    xspec = pl.BlockSpec((RB, 2, H), lambda i: (i, 0, 0)) if xfirst else stacked
    if final:
        out_spec = pl.BlockSpec((RB, D), lambda i: (i, 0))
        out_shape = jax.ShapeDtypeStruct((N, D), jnp.float32)
    else:
        out_spec = stacked
        out_shape = jax.ShapeDtypeStruct((2, N, H), jnp.float32)
    return pl.pallas_call(
        functools.partial(_mlp_body, final=final, xfirst=xfirst),
        grid=(N // RB,),
        in_specs=[stacked, xspec, full, vec, full, vec, vec, vec],
        out_specs=out_spec,
        out_shape=out_shape,
    )


_mlp_first = _make_mlp(final=False, xfirst=True)
_mlp_stacked = _make_mlp(final=False)
_mlp_final = _make_mlp(final=True)


def kernel(x, adj_t,
           W1_0, b1_0, W2_0, b2_0, g_0, be_0,
           W1_1, b1_1, W2_1, b2_1, g_1, be_1,
           W1_2, b1_2, W2_2, b2_2, g_2, be_2):
    params = [
        (W1_0, b1_0, W2_0, b2_0, g_0, be_0),
        (W1_1, b1_1, W2_1, b2_1, g_1, be_1),
        (W1_2, b1_2, W2_2, b2_2, g_2, be_2),
    ]
    npad_t = (EPAD - E) // NT
    src2d = adj_t[0].reshape(NT, E // NT)
    dst2d = adj_t[1].reshape(NT, E // NT)
    if npad_t:
        dsrc = jnp.zeros((NT, npad_t), jnp.int32)
        ddst = jnp.broadcast_to(
            N + jnp.arange(npad_t, dtype=jnp.int32), (NT, npad_t))
        src2d = jnp.concatenate([src2d, dsrc], axis=1)
        dst2d = jnp.concatenate([dst2d, ddst], axis=1)
    src16 = src2d.reshape(NT, NCHUNK, CEDGE)
    src_all = jnp.concatenate([src16, src16 + N], axis=0)
    src_all = src_all.reshape(2 * NT * 2, HCH, CEDGE)
    src0_all = jnp.concatenate([2 * src16, 2 * src16 + 1], axis=0)
    src0_all = src0_all.reshape(2 * NT * 2, HCH, CEDGE)
    dst_all = dst2d.reshape(NT * 2, HCH, CEDGE)
    zeros = jnp.zeros((ROWS_PER_TILE, H), jnp.float32)
    x2 = None
    for i in range(3):
        xs = x.reshape(2 * N, H) if i == 0 else x2
        srcs = src0_all if i == 0 else src_all
        agg3 = _sc_aggregate(xs, srcs, dst_all, zeros).reshape(2, NP_, H)
        x3 = x.reshape(N, 2, H) if i == 0 else x2.reshape(2, N, H)
        p = [params[i][0], params[i][1].reshape(1, D),
             params[i][2], params[i][3].reshape(1, D),
             params[i][4].reshape(1, D), params[i][5].reshape(1, D)]
        if i == 0:
            x2 = _mlp_first(agg3, x3, *p).reshape(2 * N, H)
        elif i == 1:
            x2 = _mlp_stacked(agg3, x3, *p).reshape(2 * N, H)
        else:
            return _mlp_final(agg3, x3, *p)


# speedup vs baseline: 1.0277x; 1.0125x over previous
import functools

import numpy as np
import jax
import jax.numpy as jnp
from jax import lax
from jax.experimental import pallas as pl
from jax.experimental.pallas import tpu as pltpu
from jax.experimental.pallas import tpu_sc as plsc

N = 10000
D = 256
E = 160000
H = 128
NT = 16
NCHUNK = 100
CEDGE = 100
HCH = NCHUNK // 2
EPAD = NT * NCHUNK * CEDGE
NP_ = 10240
ROWS_PER_TILE = NP_ // NT
RB = 5000
BN_INV = float(1.0 / np.sqrt(1.0 + 1e-5))

_sc_mesh = plsc.VectorSubcoreMesh(core_axis_name="c", subcore_axis_name="s")


@functools.partial(
    pl.kernel,
    out_type=jax.ShapeDtypeStruct((2 * NP_, H), jnp.float32),
    mesh=_sc_mesh,
    scratch_types=[
        pltpu.VMEM_SHARED((NP_, H), jnp.float32),
        pltpu.VMEM((HCH, CEDGE), jnp.int32),
        pltpu.VMEM((HCH, CEDGE), jnp.int32),
        pltpu.VMEM((CEDGE, H), jnp.float32),
        pltpu.VMEM((CEDGE, H), jnp.float32),
        pltpu.SemaphoreType.DMA,
        pltpu.SemaphoreType.DMA,
    ],
)
def _sc_aggregate(x_hbm, src_hbm, dst_hbm, zeros_hbm, out_hbm,
                  acc, src_v, dst_v, rows0, rows1, sem0, sem1):
    c = lax.axis_index("c")
    s = lax.axis_index("s")
    row0 = s * ROWS_PER_TILE
    w = c * NT + s
    pltpu.sync_copy(zeros_hbm, acc.at[pl.ds(row0, ROWS_PER_TILE)])
    plsc.subcore_barrier()

    for h in range(2):
        pltpu.sync_copy(src_hbm.at[2 * w + h], src_v)
        pltpu.sync_copy(dst_hbm.at[2 * s + h], dst_v)
        pltpu.async_copy(x_hbm.at[src_v.at[0]], rows0, sem0)

        def pair(p, carry):
            j0 = 2 * p
            pltpu.async_copy(x_hbm.at[src_v.at[j0 + 1]], rows1, sem1)
            pltpu.make_async_copy(x_hbm.at[src_v.at[j0]], rows0, sem0).wait()
            pltpu.sync_copy(rows0, acc.at[dst_v.at[j0]], add=True)

            @pl.when(p < HCH // 2 - 1)
            def _():
                pltpu.async_copy(x_hbm.at[src_v.at[j0 + 2]], rows0, sem0)

            pltpu.make_async_copy(x_hbm.at[src_v.at[j0 + 1]], rows1,
                                  sem1).wait()
            pltpu.sync_copy(rows1, acc.at[dst_v.at[j0 + 1]], add=True)
            return carry

        lax.fori_loop(0, HCH // 2, pair, 0)
    plsc.subcore_barrier()
    pltpu.sync_copy(acc.at[pl.ds(row0, ROWS_PER_TILE)],
                    out_hbm.at[pl.ds(c * NP_ + row0, ROWS_PER_TILE)])


def _mlp_body(agg_ref, x_ref, w1_ref, b1_ref, w2_ref, b2_ref, g_ref, be_ref,
              out_ref, *, final, xfirst):
    if xfirst:
        xl = x_ref[:, 0, :]
        xr = x_ref[:, 1, :]
    else:
        xl = x_ref[0]
        xr = x_ref[1]
    hl = agg_ref[0] + xl
    hr = agg_ref[1] + xr
    u = (jnp.dot(hl, w1_ref[:H, :], preferred_element_type=jnp.float32)
         + jnp.dot(hr, w1_ref[H:, :], preferred_element_type=jnp.float32)
         + b1_ref[0])
    u = jnp.maximum(u, 0.0)
    y = jnp.dot(u, w2_ref[...], preferred_element_type=jnp.float32) + b2_ref[0]
    y = jnp.maximum(y, 0.0)
    y = y * (g_ref[0] * BN_INV) + be_ref[0]
    if final:
        out_ref[...] = y
    else:
        out_ref[0] = y[:, :H]
        out_ref[1] = y[:, H:]


def _make_mlp(final, xfirst=False):
    stacked = pl.BlockSpec((2, RB, H), lambda i: (0, i, 0))
    full = pl.BlockSpec((D, D), lambda i: (0, 0))
    vec = pl.BlockSpec((1, D), lambda i: (0, 0))
    xspec = pl.BlockSpec((RB, 2, H), lambda i: (i, 0, 0)) if xfirst else stacked
    if final:
        out_spec = pl.BlockSpec((RB, D), lambda i: (i, 0))
        out_shape = jax.ShapeDtypeStruct((N, D), jnp.float32)
    else:
        out_spec = stacked
        out_shape = jax.ShapeDtypeStruct((2, N, H), jnp.float32)
    return pl.pallas_call(
        functools.partial(_mlp_body, final=final, xfirst=xfirst),
        grid=(N // RB,),
        in_specs=[stacked, xspec, full, vec, full, vec, vec, vec],
        out_specs=out_spec,
        out_shape=out_shape,
    )


_mlp_first = _make_mlp(final=False, xfirst=True)
_mlp_stacked = _make_mlp(final=False)
_mlp_final = _make_mlp(final=True)


def kernel(x, adj_t,
           W1_0, b1_0, W2_0, b2_0, g_0, be_0,
           W1_1, b1_1, W2_1, b2_1, g_1, be_1,
           W1_2, b1_2, W2_2, b2_2, g_2, be_2):
    params = [
        (W1_0, b1_0, W2_0, b2_0, g_0, be_0),
        (W1_1, b1_1, W2_1, b2_1, g_1, be_1),
        (W1_2, b1_2, W2_2, b2_2, g_2, be_2),
    ]
    npad_t = (EPAD - E) // NT
    src2d = adj_t[0].reshape(NT, E // NT)
    dst2d = adj_t[1].reshape(NT, E // NT)
    if npad_t:
        dsrc = jnp.zeros((NT, npad_t), jnp.int32)
        ddst = jnp.broadcast_to(
            N + jnp.arange(npad_t, dtype=jnp.int32), (NT, npad_t))
        src2d = jnp.concatenate([src2d, dsrc], axis=1)
        dst2d = jnp.concatenate([dst2d, ddst], axis=1)
    src16 = src2d.reshape(NT, NCHUNK, CEDGE)
    src_all = jnp.concatenate([src16, src16 + N], axis=0)
    src_all = src_all.reshape(2 * NT * 2, HCH, CEDGE)
    src0_all = jnp.concatenate([2 * src16, 2 * src16 + 1], axis=0)
    src0_all = src0_all.reshape(2 * NT * 2, HCH, CEDGE)
    dst_all = dst2d.reshape(NT * 2, HCH, CEDGE)
    zeros = jnp.zeros((ROWS_PER_TILE, H), jnp.float32)
    x2 = None
    for i in range(3):
        xs = x.reshape(2 * N, H) if i == 0 else x2
        srcs = src0_all if i == 0 else src_all
        agg3 = _sc_aggregate(xs, srcs, dst_all, zeros).reshape(2, NP_, H)
        x3 = x.reshape(N, 2, H) if i == 0 else x2.reshape(2, N, H)
        p = [params[i][0], params[i][1].reshape(1, D),
             params[i][2], params[i][3].reshape(1, D),
             params[i][4].reshape(1, D), params[i][5].reshape(1, D)]
        if i == 0:
            x2 = _mlp_first(agg3, x3, *p).reshape(2 * N, H)
        elif i == 1:
            x2 = _mlp_stacked(agg3, x3, *p).reshape(2 * N, H)
        else:
            return _mlp_final(agg3, x3, *p)
